# baseline (speedup 1.0000x reference)
import functools

import jax
import jax.numpy as jnp
from jax import lax
from jax.experimental import pallas as pl
from jax.experimental.pallas import tpu as pltpu
from jax.experimental.pallas import tpu_sc as plsc

_B = 16384
_K = 16
_INFO = plsc.get_sparse_core_info()
_NC = _INFO.num_cores
_NS = _INFO.num_subcores
_NW = _NC * _NS
_BPW = _B // _NW
_CHUNK = 128
_NCHUNK = _BPW // _CHUNK
_XROWS = 2 * _BPW // 128

_mesh = plsc.VectorSubcoreMesh(core_axis_name="c", subcore_axis_name="s")


@functools.partial(
    pl.kernel,
    mesh=_mesh,
    compiler_params=pltpu.CompilerParams(
        needs_layout_passes=False, use_tc_tiling_on_sc=False),
    out_type=jax.ShapeDtypeStruct((_B,), jnp.float32),
    scratch_types=[
        pltpu.VMEM((_XROWS, 128), jnp.int32),
        pltpu.VMEM((_NCHUNK, _CHUNK), jnp.int32),
        pltpu.VMEM((_NCHUNK, _CHUNK), jnp.int32),
        pltpu.VMEM((_BPW, _K), jnp.float32),
        pltpu.VMEM((_BPW, _K), jnp.float32),
        pltpu.VMEM((_BPW,), jnp.float32),
        pltpu.SemaphoreType.DMA,
    ],
)
def _mf_dot(x_hbm, w_hbm, h_hbm, out_hbm,
            x_v, uidx_v, iidx_v, u_v, v_v, o_v, sem):
    wid = lax.axis_index("s") * _NC + lax.axis_index("c")
    base = wid * _BPW

    pltpu.sync_copy(x_hbm.at[pl.ds(wid * _XROWS, _XROWS)], x_v)

    lanes = lax.iota(jnp.int32, 16)
    for c in range(_NCHUNK):
        for j in range(8):
            row = jnp.full((16,), 2 * c + j // 4, jnp.int32)
            col = 32 * (j % 4) + 2 * lanes
            uidx_v[c, pl.ds(16 * j, 16)] = plsc.load_gather(x_v, [row, col])
            iidx_v[c, pl.ds(16 * j, 16)] = plsc.load_gather(x_v, [row, col + 1])

    copies = []
    for c in range(_NCHUNK):
        copies.append(pltpu.async_copy(
            w_hbm.at[uidx_v.at[c]], u_v.at[pl.ds(c * _CHUNK, _CHUNK)], sem))
        copies.append(pltpu.async_copy(
            h_hbm.at[iidx_v.at[c]], v_v.at[pl.ds(c * _CHUNK, _CHUNK)], sem))
    for cp in copies:
        cp.wait()

    def body(g, carry):
        acc = jnp.zeros((16,), jnp.float32)
        for j in range(16):
            row = g * 16 + j
            prod = u_v[row] * v_v[row]
            acc = jnp.where(lanes == j, jnp.sum(prod), acc)
        o_v[pl.ds(g * 16, 16)] = acc
        return carry

    lax.fori_loop(0, _BPW // 16, body, 0)

    pltpu.sync_copy(o_v, out_hbm.at[pl.ds(base, _BPW)])


def kernel(x, W, H):
    xr = x.astype(jnp.int32).reshape(_B * 2 // 128, 128)
    return _mf_dot(xr, W, H)


# speedup vs baseline: 5.9092x; 5.9092x over previous
import functools

import jax
import jax.numpy as jnp
from jax import lax
from jax.experimental import pallas as pl
from jax.experimental.pallas import tpu as pltpu
from jax.experimental.pallas import tpu_sc as plsc

_B = 16384
_K = 16
_INFO = plsc.get_sparse_core_info()
_NC = _INFO.num_cores
_NS = _INFO.num_subcores
_NW = _NC * _NS
_BPW = _B // _NW
_NB = 16
_NCHUNK = _BPW // _NB

_mesh = plsc.VectorSubcoreMesh(core_axis_name="c", subcore_axis_name="s")


@functools.partial(
    pl.kernel,
    mesh=_mesh,
    compiler_params=pltpu.CompilerParams(
        needs_layout_passes=False, use_tc_tiling_on_sc=True),
    out_type=jax.ShapeDtypeStruct((_B,), jnp.float32),
    scratch_types=[
        pltpu.VMEM((_BPW,), jnp.int32),
        pltpu.VMEM((_BPW,), jnp.int32),
        pltpu.VMEM((_NB, _K, 128), jnp.float32),
        pltpu.VMEM((_NB, _K, 128), jnp.float32),
        pltpu.VMEM((_BPW,), jnp.float32),
        pltpu.SemaphoreType.DMA,
    ],
)
def _mf_dot(xt_hbm, wt_hbm, ht_hbm, out_hbm,
            ux_v, ix_v, wslab_v, hslab_v, o_v, sem):
    wid = lax.axis_index("s") * _NC + lax.axis_index("c")
    base = wid * _BPW

    pltpu.sync_copy(xt_hbm.at[0, pl.ds(base, _BPW)], ux_v)
    pltpu.sync_copy(xt_hbm.at[1, pl.ds(base, _BPW)], ix_v)

    lanes = lax.iota(jnp.int32, 16)

    def body(c, carry):
        uvec = ux_v[pl.ds(c * _NB, _NB)]
        ivec = ix_v[pl.ds(c * _NB, _NB)]
        copies = []
        for j in range(_NB):
            u = uvec[j]
            i = ivec[j]
            uoff = pl.multiple_of((u >> 7) * 128, 128)
            ioff = pl.multiple_of((i >> 7) * 128, 128)
            copies.append(pltpu.async_copy(
                wt_hbm.at[:, pl.ds(uoff, 128)], wslab_v.at[j], sem))
            copies.append(pltpu.async_copy(
                ht_hbm.at[:, pl.ds(ioff, 128)], hslab_v.at[j], sem))
        for cp in copies:
            cp.wait()
        acc = jnp.zeros((16,), jnp.float32)
        for j in range(_NB):
            ucol = jnp.full((16,), uvec[j] & 127, jnp.int32)
            icol = jnp.full((16,), ivec[j] & 127, jnp.int32)
            jj = jnp.full((16,), j, jnp.int32)
            wcol = plsc.load_gather(wslab_v, [jj, lanes, ucol])
            hcol = plsc.load_gather(hslab_v, [jj, lanes, icol])
            acc = jnp.where(lanes == j, jnp.sum(wcol * hcol), acc)
        o_v[pl.ds(c * _NB, _NB)] = acc
        return carry

    lax.fori_loop(0, _NCHUNK, body, 0)

    pltpu.sync_copy(o_v, out_hbm.at[pl.ds(base, _BPW)])


def kernel(x, W, H):
    return _mf_dot(x.astype(jnp.int32).T, W.T, H.T)


# speedup vs baseline: 6.0293x; 1.0203x over previous
import functools

import jax
import jax.numpy as jnp
from jax import lax
from jax.experimental import pallas as pl
from jax.experimental.pallas import tpu as pltpu
from jax.experimental.pallas import tpu_sc as plsc

_B = 16384
_K = 16
_INFO = plsc.get_sparse_core_info()
_NC = _INFO.num_cores
_NS = _INFO.num_subcores
_NW = _NC * _NS
_BPW = _B // _NW
_NB = 8
_NPAIR = _BPW // (2 * _NB)

_mesh = plsc.VectorSubcoreMesh(core_axis_name="c", subcore_axis_name="s")


@functools.partial(
    pl.kernel,
    mesh=_mesh,
    compiler_params=pltpu.CompilerParams(
        needs_layout_passes=False, use_tc_tiling_on_sc=True),
    out_type=jax.ShapeDtypeStruct((_B,), jnp.float32),
    scratch_types=[
        pltpu.VMEM((_BPW,), jnp.int32),
        pltpu.VMEM((_BPW,), jnp.int32),
        pltpu.VMEM((2, _NB, _K, 128), jnp.float32),
        pltpu.VMEM((2, _NB, _K, 128), jnp.float32),
        pltpu.VMEM((_BPW,), jnp.float32),
        pltpu.SemaphoreType.DMA,
        pltpu.SemaphoreType.DMA,
    ],
)
def _mf_dot(xt_hbm, wt_hbm, ht_hbm, out_hbm,
            ux_v, ix_v, wslab_v, hslab_v, o_v, sem0, sem1):
    wid = lax.axis_index("s") * _NC + lax.axis_index("c")
    base = wid * _BPW

    pltpu.sync_copy(xt_hbm.at[0, pl.ds(base, _BPW)], ux_v)
    pltpu.sync_copy(xt_hbm.at[1, pl.ds(base, _BPW)], ix_v)

    lanes = lax.iota(jnp.int32, 16)

    def fire(uvec, ivec, lane_off, par, sem):
        for j in range(_NB):
            u = uvec[lane_off + j]
            i = ivec[lane_off + j]
            uoff = pl.multiple_of((u >> 7) * 128, 128)
            ioff = pl.multiple_of((i >> 7) * 128, 128)
            pltpu.async_copy(
                wt_hbm.at[:, pl.ds(uoff, 128)], wslab_v.at[par, j], sem)
            pltpu.async_copy(
                ht_hbm.at[:, pl.ds(ioff, 128)], hslab_v.at[par, j], sem)

    def drain(par, sem):
        for j in range(_NB):
            pltpu.make_async_copy(
                wt_hbm.at[:, pl.ds(0, 128)], wslab_v.at[par, j], sem).wait()
            pltpu.make_async_copy(
                ht_hbm.at[:, pl.ds(0, 128)], hslab_v.at[par, j], sem).wait()

    def reduce_group(uvec, ivec, lane_off, par, acc):
        for j in range(_NB):
            ucol = jnp.full((16,), uvec[lane_off + j] & 127, jnp.int32)
            icol = jnp.full((16,), ivec[lane_off + j] & 127, jnp.int32)
            jj = jnp.full((16,), j, jnp.int32)
            pp = jnp.full((16,), par, jnp.int32)
            wcol = plsc.load_gather(wslab_v, [pp, jj, lanes, ucol])
            hcol = plsc.load_gather(hslab_v, [pp, jj, lanes, icol])
            acc = jnp.where(lanes == lane_off + j, jnp.sum(wcol * hcol), acc)
        return acc

    uv0 = ux_v[pl.ds(0, 16)]
    iv0 = ix_v[pl.ds(0, 16)]
    fire(uv0, iv0, 0, 0, sem0)

    def body(p, carry):
        s = p * 16
        uvec = ux_v[pl.ds(s, 16)]
        ivec = ix_v[pl.ds(s, 16)]
        fire(uvec, ivec, _NB, 1, sem1)
        drain(0, sem0)
        acc = reduce_group(uvec, ivec, 0, 0, jnp.zeros((16,), jnp.float32))
        @pl.when(p + 1 < _NPAIR)
        def _():
            unext = ux_v[pl.ds(s + 16, 16)]
            inext = ix_v[pl.ds(s + 16, 16)]
            fire(unext, inext, 0, 0, sem0)
        drain(1, sem1)
        acc = reduce_group(uvec, ivec, _NB, 1, acc)
        o_v[pl.ds(s, 16)] = acc
        return carry

    lax.fori_loop(0, _NPAIR, body, 0)

    pltpu.sync_copy(o_v, out_hbm.at[pl.ds(base, _BPW)])


def kernel(x, W, H):
    return _mf_dot(x.astype(jnp.int32).T, W.T, H.T)


# speedup vs baseline: 6.0389x; 1.0016x over previous
import functools

import jax
import jax.numpy as jnp
from jax import lax
from jax.experimental import pallas as pl
from jax.experimental.pallas import tpu as pltpu
from jax.experimental.pallas import tpu_sc as plsc

_B = 16384
_K = 16
_INFO = plsc.get_sparse_core_info()
_NC = _INFO.num_cores
_NS = _INFO.num_subcores
_NW = _NC * _NS
_BPW = _B // _NW
_NB = 8
_NPAIR = _BPW // (2 * _NB)

_mesh = plsc.VectorSubcoreMesh(core_axis_name="c", subcore_axis_name="s")


@functools.partial(
    pl.kernel,
    mesh=_mesh,
    compiler_params=pltpu.CompilerParams(
        needs_layout_passes=False, use_tc_tiling_on_sc=True),
    out_type=(
        jax.ShapeDtypeStruct((_B,), jnp.float32),
        jax.ShapeDtypeStruct((_NB, _K, 128), jnp.float32),
    ),
    scratch_types=[
        pltpu.VMEM((_BPW,), jnp.int32),
        pltpu.VMEM((_BPW,), jnp.int32),
        pltpu.VMEM((2, _NB, _K, 128), jnp.float32),
        pltpu.VMEM((2, _NB, _K, 128), jnp.float32),
        pltpu.VMEM((_BPW,), jnp.float32),
        pltpu.SemaphoreType.DMA,
        pltpu.SemaphoreType.DMA,
    ],
)
def _mf_dot(xt_hbm, wt_hbm, ht_hbm, out_hbm, dummy_hbm,
            ux_v, ix_v, wslab_v, hslab_v, o_v, sem0, sem1):
    wid = lax.axis_index("s") * _NC + lax.axis_index("c")
    base = wid * _BPW

    pltpu.sync_copy(xt_hbm.at[0, pl.ds(base, _BPW)], ux_v)
    pltpu.sync_copy(xt_hbm.at[1, pl.ds(base, _BPW)], ix_v)

    lanes = lax.iota(jnp.int32, 16)

    def fire(uvec, ivec, lane_off, par, sem):
        for j in range(_NB):
            u = uvec[lane_off + j]
            i = ivec[lane_off + j]
            uoff = pl.multiple_of((u >> 7) * 128, 128)
            ioff = pl.multiple_of((i >> 7) * 128, 128)
            pltpu.async_copy(
                wt_hbm.at[:, pl.ds(uoff, 128)], wslab_v.at[par, j], sem)
            pltpu.async_copy(
                ht_hbm.at[:, pl.ds(ioff, 128)], hslab_v.at[par, j], sem)

    def drain(par, sem):
        pltpu.make_async_copy(dummy_hbm, wslab_v.at[par], sem).wait()
        pltpu.make_async_copy(dummy_hbm, hslab_v.at[par], sem).wait()

    def reduce_group(uvec, ivec, lane_off, par, acc):
        for j in range(_NB):
            ucol = jnp.full((16,), uvec[lane_off + j] & 127, jnp.int32)
            icol = jnp.full((16,), ivec[lane_off + j] & 127, jnp.int32)
            jj = jnp.full((16,), j, jnp.int32)
            pp = jnp.full((16,), par, jnp.int32)
            wcol = plsc.load_gather(wslab_v, [pp, jj, lanes, ucol])
            hcol = plsc.load_gather(hslab_v, [pp, jj, lanes, icol])
            acc = jnp.where(lanes == lane_off + j, jnp.sum(wcol * hcol), acc)
        return acc

    uv0 = ux_v[pl.ds(0, 16)]
    iv0 = ix_v[pl.ds(0, 16)]
    fire(uv0, iv0, 0, 0, sem0)

    def body(p, carry):
        s = p * 16
        uvec = ux_v[pl.ds(s, 16)]
        ivec = ix_v[pl.ds(s, 16)]
        fire(uvec, ivec, _NB, 1, sem1)
        drain(0, sem0)
        acc = reduce_group(uvec, ivec, 0, 0, jnp.zeros((16,), jnp.float32))
        @pl.when(p + 1 < _NPAIR)
        def _():
            unext = ux_v[pl.ds(s + 16, 16)]
            inext = ix_v[pl.ds(s + 16, 16)]
            fire(unext, inext, 0, 0, sem0)
        drain(1, sem1)
        acc = reduce_group(uvec, ivec, _NB, 1, acc)
        o_v[pl.ds(s, 16)] = acc
        return carry

    lax.fori_loop(0, _NPAIR, body, 0)

    pltpu.sync_copy(o_v, out_hbm.at[pl.ds(base, _BPW)])


def kernel(x, W, H):
    out, _ = _mf_dot(x.astype(jnp.int32).T, W.T, H.T)
    return out
